# Initial kernel scaffold; baseline (speedup 1.0000x reference)
import jax
import jax.numpy as jnp
from jax.experimental import pallas as pl


def kernel(edge_index, features, W1, b1, W2, b2, Wc):
    raise NotImplementedError("write your pallas kernel here")


# speedup vs baseline: 8.0165x; 8.0165x over previous
import functools

import jax
import jax.numpy as jnp
from jax import lax
from jax.experimental import pallas as pl
from jax.experimental.pallas import tpu as pltpu
from jax.experimental.pallas import tpu_sc as plsc

N_NODES = 10000
N_EDGES = 160000
DF = 256
DH = 512
D = 128

NC, NS = 2, 16
NW = NC * NS
EPW = N_EDGES // NW
CHUNK = 125
NCHUNK = EPW // CHUNK
ROWS_PT = 624
ROWS_LAST = N_NODES - 15 * ROWS_PT

MLP_BLK = 1000
MLP_GRID = N_NODES // MLP_BLK


def _mlp_body(f_ref, w1_ref, b1_ref, w2_ref, b2_ref, wc_ref, o_ref):
    f = f_ref[...]
    h = lax.dot_general(f, w1_ref[...], (((1,), (1,)), ((), ())),
                        preferred_element_type=jnp.float32)
    h = h + b1_ref[...]
    h = jnp.where(h >= 0, h, 0.01 * h)
    x = lax.dot_general(h, w2_ref[...], (((1,), (1,)), ((), ())),
                        preferred_element_type=jnp.float32)
    x = x + b2_ref[...]
    o_ref[...] = jnp.dot(x, wc_ref[...], preferred_element_type=jnp.float32)


def _mlp(features, W1, b1, W2, b2, Wc):
    return pl.pallas_call(
        _mlp_body,
        grid=(MLP_GRID,),
        in_specs=[
            pl.BlockSpec((MLP_BLK, DF), lambda i: (i, 0)),
            pl.BlockSpec((DH, DF), lambda i: (0, 0)),
            pl.BlockSpec((1, DH), lambda i: (0, 0)),
            pl.BlockSpec((D, DH), lambda i: (0, 0)),
            pl.BlockSpec((1, D), lambda i: (0, 0)),
            pl.BlockSpec((D, D), lambda i: (0, 0)),
        ],
        out_specs=pl.BlockSpec((MLP_BLK, D), lambda i: (i, 0)),
        out_shape=jax.ShapeDtypeStruct((N_NODES, D), jnp.float32),
    )(features, W1, b1, W2, b2, Wc)


@functools.cache
def _build_gcn_scatter():
  mesh = plsc.VectorSubcoreMesh(
      core_axis_name="c", subcore_axis_name="s", num_cores=NC, num_subcores=NS)

  @functools.partial(
      pl.kernel,
      out_type=jax.ShapeDtypeStruct((NC, N_NODES, D), jnp.float32),
      mesh=mesh,
      scratch_types=[
          pltpu.VMEM((NCHUNK, CHUNK), jnp.int32),
          pltpu.VMEM((NCHUNK, CHUNK), jnp.int32),
          pltpu.VMEM((CHUNK, D), jnp.float32),
          pltpu.VMEM((CHUNK, D), jnp.float32),
          pltpu.VMEM_SHARED((N_NODES, D), jnp.float32),
          pltpu.SemaphoreType.DMA,
          pltpu.SemaphoreType.DMA,
      ],
  )
  def _gcn_scatter(xlin_hbm, src_hbm, dst_hbm, zeros_hbm, out_hbm,
                   src_v, dst_v, buf0, buf1, acc, sem0, sem1):
    c = lax.axis_index("c")
    s = lax.axis_index("s")
    w = c * NS + s

    @pl.when(s < NS - 1)
    def _():
      pltpu.sync_copy(zeros_hbm.at[pl.ds(s * ROWS_PT, ROWS_PT)],
                      acc.at[pl.ds(s * ROWS_PT, ROWS_PT)])

    @pl.when(s == NS - 1)
    def _():
      pltpu.sync_copy(zeros_hbm.at[pl.ds(15 * ROWS_PT, ROWS_LAST)],
                      acc.at[pl.ds(15 * ROWS_PT, ROWS_LAST)])

    pltpu.sync_copy(src_hbm.at[w], src_v)
    pltpu.sync_copy(dst_hbm.at[w], dst_v)
    plsc.subcore_barrier()

    def start_gather(j, buf, sem):
      pltpu.async_copy(xlin_hbm.at[src_v.at[j]], buf, sem)

    def wait_gather(j, buf, sem):
      pltpu.make_async_copy(xlin_hbm.at[src_v.at[j]], buf, sem).wait()

    start_gather(0, buf0, sem0)

    def body(i, carry):
      j0 = 2 * i
      start_gather(j0 + 1, buf1, sem1)
      wait_gather(j0, buf0, sem0)
      pltpu.sync_copy(buf0, acc.at[dst_v.at[j0]], add=True)

      @pl.when(i < NCHUNK // 2 - 1)
      def _():
        start_gather(j0 + 2, buf0, sem0)

      wait_gather(j0 + 1, buf1, sem1)
      pltpu.sync_copy(buf1, acc.at[dst_v.at[j0 + 1]], add=True)
      return carry

    lax.fori_loop(0, NCHUNK // 2, body, 0)

    plsc.subcore_barrier()

    @pl.when(s < NS - 1)
    def _():
      pltpu.sync_copy(acc.at[pl.ds(s * ROWS_PT, ROWS_PT)],
                      out_hbm.at[c, pl.ds(s * ROWS_PT, ROWS_PT)])

    @pl.when(s == NS - 1)
    def _():
      pltpu.sync_copy(acc.at[pl.ds(15 * ROWS_PT, ROWS_LAST)],
                      out_hbm.at[c, pl.ds(15 * ROWS_PT, ROWS_LAST)])

  return _gcn_scatter


def _add_body(a_ref, b_ref, o_ref):
    o_ref[...] = a_ref[...] + b_ref[...]


def _add(a, b):
    return pl.pallas_call(
        _add_body,
        grid=(MLP_GRID,),
        in_specs=[
            pl.BlockSpec((MLP_BLK, D), lambda i: (i, 0)),
            pl.BlockSpec((MLP_BLK, D), lambda i: (i, 0)),
        ],
        out_specs=pl.BlockSpec((MLP_BLK, D), lambda i: (i, 0)),
        out_shape=jax.ShapeDtypeStruct((N_NODES, D), jnp.float32),
    )(a, b)


def kernel(edge_index, features, W1, b1, W2, b2, Wc):
    x_lin = _mlp(features, W1, b1.reshape(1, DH), W2, b2.reshape(1, D), Wc)
    src = edge_index[0].reshape(NW, NCHUNK, CHUNK)
    dst = edge_index[1].reshape(NW, NCHUNK, CHUNK)
    zeros = jnp.zeros((N_NODES, D), jnp.float32)
    partials = _build_gcn_scatter()(x_lin, src, dst, zeros)
    return _add(partials[0], partials[1])


# speedup vs baseline: 9.8499x; 1.2287x over previous
import functools

import jax
import jax.numpy as jnp
import numpy as np
from jax import lax
from jax.experimental import pallas as pl
from jax.experimental.pallas import tpu as pltpu
from jax.experimental.pallas import tpu_sc as plsc

N_NODES = 10000
N_EDGES = 160000
DF = 256
DH = 512
D = 128

NC, NS = 2, 16
NW = NC * NS
CHUNK = 128
NCHUNK = 39
NXTRA = N_EDGES // CHUNK - NW * NCHUNK
ROWS_PT = 624
ROWS_LAST = N_NODES - 15 * ROWS_PT
ZROWS = 16

MLP_BLK = 2000
MLP_GRID = N_NODES // MLP_BLK
ADD_BLK = 2000
ADD_GRID = N_NODES // ADD_BLK


def _mlp_body(f_ref, w1_ref, b1_ref, w2_ref, b2_ref, wc_ref, o_ref, z_ref):
    z_ref[...] = jnp.zeros_like(z_ref)
    f = f_ref[...].astype(jnp.bfloat16)
    h = lax.dot_general(f, w1_ref[...].astype(jnp.bfloat16),
                        (((1,), (1,)), ((), ())),
                        preferred_element_type=jnp.float32)
    h = h + b1_ref[...][None, :]
    h = jnp.where(h >= 0, h, 0.01 * h)
    x = lax.dot_general(h.astype(jnp.bfloat16), w2_ref[...].astype(jnp.bfloat16),
                        (((1,), (1,)), ((), ())),
                        preferred_element_type=jnp.float32)
    x = x + b2_ref[...][None, :]
    o_ref[...] = jnp.dot(x, wc_ref[...], preferred_element_type=jnp.float32)


def _mlp(features, W1, b1, W2, b2, Wc):
    return pl.pallas_call(
        _mlp_body,
        grid=(MLP_GRID,),
        in_specs=[
            pl.BlockSpec((MLP_BLK, DF), lambda i: (i, 0)),
            pl.BlockSpec((DH, DF), lambda i: (0, 0)),
            pl.BlockSpec((DH,), lambda i: (0,)),
            pl.BlockSpec((D, DH), lambda i: (0, 0)),
            pl.BlockSpec((D,), lambda i: (0,)),
            pl.BlockSpec((D, D), lambda i: (0, 0)),
        ],
        out_specs=[pl.BlockSpec((MLP_BLK, D), lambda i: (i, 0)),
                   pl.BlockSpec((ROWS_LAST // MLP_GRID, D), lambda i: (i, 0))],
        out_shape=[jax.ShapeDtypeStruct((N_NODES, D), jnp.float32),
                   jax.ShapeDtypeStruct((ROWS_LAST, D), jnp.float32)],
    )(features, W1, b1, W2, b2, Wc)


@functools.cache
def _build_gcn_scatter():
  mesh = plsc.VectorSubcoreMesh(
      core_axis_name="c", subcore_axis_name="s", num_cores=NC, num_subcores=NS)

  nbuf = 2
  nck = NCHUNK + 1

  @functools.partial(
      pl.kernel,
      out_type=[jax.ShapeDtypeStruct((N_NODES, D), jnp.float32),
                jax.ShapeDtypeStruct((N_NODES, D), jnp.float32)],
      mesh=mesh,
      scratch_types=[
          pltpu.VMEM((NCHUNK * CHUNK,), jnp.int32),
          pltpu.VMEM((NCHUNK * CHUNK,), jnp.int32),
          pltpu.VMEM((CHUNK,), jnp.int32),
          pltpu.VMEM((CHUNK,), jnp.int32),
          [pltpu.VMEM((CHUNK, D), jnp.float32) for _ in range(nbuf)],
          pltpu.VMEM_SHARED((N_NODES, D), jnp.float32),
          [pltpu.SemaphoreType.DMA for _ in range(nbuf)],
      ],
  )
  def _gcn_scatter(ei_hbm, xlin_hbm, zeros_hbm, out0_hbm, out1_hbm,
                   src_v, dst_v, xsrc_v, xdst_v, bufs, acc, gsems):
    c = lax.axis_index("c")
    s = lax.axis_index("s")
    w = c * NS + s

    @pl.when(s < NS - 1)
    def _():
      pltpu.sync_copy(zeros_hbm.at[pl.ds(0, ROWS_PT)],
                      acc.at[pl.ds(s * ROWS_PT, ROWS_PT)])

    @pl.when(s == NS - 1)
    def _():
      pltpu.sync_copy(zeros_hbm.at[pl.ds(0, ROWS_LAST)],
                      acc.at[pl.ds(15 * ROWS_PT, ROWS_LAST)])

    off = w * (NCHUNK * CHUNK)
    pltpu.sync_copy(ei_hbm.at[0, pl.ds(off, NCHUNK * CHUNK)], src_v)
    pltpu.sync_copy(ei_hbm.at[1, pl.ds(off, NCHUNK * CHUNK)], dst_v)

    @pl.when(s == 0)
    def _():
      xoff = (NW * NCHUNK + c) * CHUNK
      pltpu.sync_copy(ei_hbm.at[0, pl.ds(xoff, CHUNK)], xsrc_v)
      pltpu.sync_copy(ei_hbm.at[1, pl.ds(xoff, CHUNK)], xdst_v)

    def sidx(j):
      return src_v.at[pl.ds(j * CHUNK, CHUNK)]

    def didx(j):
      return dst_v.at[pl.ds(j * CHUNK, CHUNK)]

    def start_gather_ref(iref, b):
      pltpu.async_copy(xlin_hbm.at[iref], bufs[b], gsems[b])

    def wait_gather_ref(iref, b):
      pltpu.make_async_copy(xlin_hbm.at[iref], bufs[b], gsems[b]).wait()

    def scatter_ref(iref, b):
      pltpu.sync_copy(bufs[b], acc.at[iref], add=True)

    def start_gather(j, b):
      start_gather_ref(sidx(j), b)

    def wait_gather(j, b):
      wait_gather_ref(sidx(j), b)

    def scatter(j, b):
      scatter_ref(didx(j), b)

    for b in range(nbuf):
      start_gather(b, b)

    plsc.subcore_barrier()

    ng = (NCHUNK - 1) // nbuf

    def body(g, carry):
      for b in range(nbuf):
        j = nbuf * g + b
        wait_gather(j, b)
        scatter(j, b)

        @pl.when(g < ng - 1)
        def _():
          start_gather(j + nbuf, b)
      return carry

    lax.fori_loop(0, ng, body, 0)

    start_gather(NCHUNK - 1, 0)

    @pl.when(s == 0)
    def _():
      start_gather_ref(xsrc_v, 1)

    wait_gather(NCHUNK - 1, 0)
    scatter(NCHUNK - 1, 0)

    @pl.when(s == 0)
    def _():
      wait_gather_ref(xsrc_v, 1)
      scatter_ref(xdst_v, 1)

    plsc.subcore_barrier()

    def writeout(out_hbm):
      @pl.when(s < NS - 1)
      def _():
        pltpu.sync_copy(acc.at[pl.ds(s * ROWS_PT, ROWS_PT)],
                        out_hbm.at[pl.ds(s * ROWS_PT, ROWS_PT)])

      @pl.when(s == NS - 1)
      def _():
        pltpu.sync_copy(acc.at[pl.ds(15 * ROWS_PT, ROWS_LAST)],
                        out_hbm.at[pl.ds(15 * ROWS_PT, ROWS_LAST)])

    @pl.when(c == 0)
    def _():
      writeout(out0_hbm)

    @pl.when(c == 1)
    def _():
      writeout(out1_hbm)

  return _gcn_scatter


def _add_body(a_ref, b_ref, o_ref):
    o_ref[...] = a_ref[...] + b_ref[...]


def _add(a, b):
    return pl.pallas_call(
        _add_body,
        grid=(ADD_GRID,),
        in_specs=[
            pl.BlockSpec((ADD_BLK, D), lambda i: (i, 0)),
            pl.BlockSpec((ADD_BLK, D), lambda i: (i, 0)),
        ],
        out_specs=pl.BlockSpec((ADD_BLK, D), lambda i: (i, 0)),
        out_shape=jax.ShapeDtypeStruct((N_NODES, D), jnp.float32),
    )(a, b)


def kernel(edge_index, features, W1, b1, W2, b2, Wc):
    x_lin, zeros = _mlp(features, W1, b1, W2, b2, Wc)
    p0, p1 = _build_gcn_scatter()(edge_index, x_lin, zeros)
    return _add(p0, p1)


# speedup vs baseline: 10.1760x; 1.0331x over previous
import functools

import jax
import jax.numpy as jnp
import numpy as np
from jax import lax
from jax.experimental import pallas as pl
from jax.experimental.pallas import tpu as pltpu
from jax.experimental.pallas import tpu_sc as plsc

N_NODES = 10000
N_EDGES = 160000
DF = 256
DH = 512
D = 128

NC, NS = 2, 16
NW = NC * NS
CHUNK = 128
NCHUNK = 39
NXTRA = N_EDGES // CHUNK - NW * NCHUNK
ROWS_PT = 624
ROWS_LAST = N_NODES - 15 * ROWS_PT
ZROWS = 16

MLP_BLK = 2000
MLP_GRID = N_NODES // MLP_BLK
ADD_BLK = 2000
ADD_GRID = N_NODES // ADD_BLK


def _mlp_body(f_ref, w1_ref, b1_ref, w2_ref, b2_ref, wc_ref, o_ref, z_ref):
    z_ref[...] = jnp.zeros_like(z_ref)
    f = f_ref[...].astype(jnp.bfloat16)
    h = lax.dot_general(f, w1_ref[...].astype(jnp.bfloat16),
                        (((1,), (1,)), ((), ())),
                        preferred_element_type=jnp.float32)
    h = h + b1_ref[...][None, :]
    h = jnp.where(h >= 0, h, 0.01 * h)
    x = lax.dot_general(h.astype(jnp.bfloat16), w2_ref[...].astype(jnp.bfloat16),
                        (((1,), (1,)), ((), ())),
                        preferred_element_type=jnp.float32)
    x = x + b2_ref[...][None, :]
    o_ref[...] = jnp.dot(x, wc_ref[...], preferred_element_type=jnp.float32)


def _mlp(features, W1, b1, W2, b2, Wc):
    return pl.pallas_call(
        _mlp_body,
        grid=(MLP_GRID,),
        in_specs=[
            pl.BlockSpec((MLP_BLK, DF), lambda i: (i, 0)),
            pl.BlockSpec((DH, DF), lambda i: (0, 0)),
            pl.BlockSpec((DH,), lambda i: (0,)),
            pl.BlockSpec((D, DH), lambda i: (0, 0)),
            pl.BlockSpec((D,), lambda i: (0,)),
            pl.BlockSpec((D, D), lambda i: (0, 0)),
        ],
        out_specs=[pl.BlockSpec((MLP_BLK, D), lambda i: (i, 0)),
                   pl.BlockSpec((ROWS_LAST // MLP_GRID, D), lambda i: (i, 0))],
        out_shape=[jax.ShapeDtypeStruct((N_NODES, D), jnp.float32),
                   jax.ShapeDtypeStruct((ROWS_LAST, D), jnp.float32)],
    )(features, W1, b1, W2, b2, Wc)


@functools.cache
def _build_gcn_scatter():
  mesh = plsc.VectorSubcoreMesh(
      core_axis_name="c", subcore_axis_name="s", num_cores=NC, num_subcores=NS)

  nbuf = 2
  nck = NCHUNK + 1

  @functools.partial(
      pl.kernel,
      out_type=[jax.ShapeDtypeStruct((N_NODES, D), jnp.float32),
                jax.ShapeDtypeStruct((N_NODES, D), jnp.float32)],
      mesh=mesh,
      scratch_types=[
          pltpu.VMEM((NCHUNK * CHUNK,), jnp.int32),
          pltpu.VMEM((NCHUNK * CHUNK,), jnp.int32),
          pltpu.VMEM((CHUNK,), jnp.int32),
          pltpu.VMEM((CHUNK,), jnp.int32),
          [pltpu.VMEM((CHUNK, D), jnp.float32) for _ in range(nbuf)],
          pltpu.VMEM_SHARED((N_NODES, D), jnp.float32),
          [pltpu.SemaphoreType.DMA for _ in range(nbuf)],
      ],
  )
  def _gcn_scatter(ei_hbm, xlin_hbm, zeros_hbm, out0_hbm, out1_hbm,
                   src_v, dst_v, xsrc_v, xdst_v, bufs, acc, gsems):
    c = lax.axis_index("c")
    s = lax.axis_index("s")
    w = c * NS + s

    off = w * (NCHUNK * CHUNK)
    pltpu.sync_copy(ei_hbm.at[0, pl.ds(off, NCHUNK * CHUNK)], src_v)
    pltpu.sync_copy(ei_hbm.at[1, pl.ds(off, NCHUNK * CHUNK)], dst_v)

    @pl.when(s == 0)
    def _():
      xoff = (NW * NCHUNK + c) * CHUNK
      pltpu.sync_copy(ei_hbm.at[0, pl.ds(xoff, CHUNK)], xsrc_v)
      pltpu.sync_copy(ei_hbm.at[1, pl.ds(xoff, CHUNK)], xdst_v)

    def sidx(j):
      return src_v.at[pl.ds(j * CHUNK, CHUNK)]

    def didx(j):
      return dst_v.at[pl.ds(j * CHUNK, CHUNK)]

    def start_gather_ref(iref, b):
      pltpu.async_copy(xlin_hbm.at[iref], bufs[b], gsems[b])

    def wait_gather_ref(iref, b):
      pltpu.make_async_copy(xlin_hbm.at[iref], bufs[b], gsems[b]).wait()

    def scatter_ref(iref, b):
      pltpu.sync_copy(bufs[b], acc.at[iref], add=True)

    def start_gather(j, b):
      start_gather_ref(sidx(j), b)

    def wait_gather(j, b):
      wait_gather_ref(sidx(j), b)

    def scatter(j, b):
      scatter_ref(didx(j), b)

    for b in range(nbuf):
      start_gather(b, b)

    @pl.when(s < NS - 1)
    def _():
      pltpu.sync_copy(zeros_hbm.at[pl.ds(0, ROWS_PT)],
                      acc.at[pl.ds(s * ROWS_PT, ROWS_PT)])

    @pl.when(s == NS - 1)
    def _():
      pltpu.sync_copy(zeros_hbm.at[pl.ds(0, ROWS_LAST)],
                      acc.at[pl.ds(15 * ROWS_PT, ROWS_LAST)])

    plsc.subcore_barrier()

    ng = (NCHUNK - 1) // nbuf

    def body(g, carry):
      for b in range(nbuf):
        j = nbuf * g + b
        wait_gather(j, b)
        scatter(j, b)

        @pl.when(g < ng - 1)
        def _():
          start_gather(j + nbuf, b)
      return carry

    lax.fori_loop(0, ng, body, 0)

    start_gather(NCHUNK - 1, 0)

    @pl.when(s == 0)
    def _():
      start_gather_ref(xsrc_v, 1)

    wait_gather(NCHUNK - 1, 0)
    scatter(NCHUNK - 1, 0)

    @pl.when(s == 0)
    def _():
      wait_gather_ref(xsrc_v, 1)
      scatter_ref(xdst_v, 1)

    plsc.subcore_barrier()

    def writeout(out_hbm):
      @pl.when(s < NS - 1)
      def _():
        pltpu.sync_copy(acc.at[pl.ds(s * ROWS_PT, ROWS_PT)],
                        out_hbm.at[pl.ds(s * ROWS_PT, ROWS_PT)])

      @pl.when(s == NS - 1)
      def _():
        pltpu.sync_copy(acc.at[pl.ds(15 * ROWS_PT, ROWS_LAST)],
                        out_hbm.at[pl.ds(15 * ROWS_PT, ROWS_LAST)])

    @pl.when(c == 0)
    def _():
      writeout(out0_hbm)

    @pl.when(c == 1)
    def _():
      writeout(out1_hbm)

  return _gcn_scatter


def _add_body(a_ref, b_ref, o_ref):
    o_ref[...] = a_ref[...] + b_ref[...]


def _add(a, b):
    return pl.pallas_call(
        _add_body,
        grid=(ADD_GRID,),
        in_specs=[
            pl.BlockSpec((ADD_BLK, D), lambda i: (i, 0)),
            pl.BlockSpec((ADD_BLK, D), lambda i: (i, 0)),
        ],
        out_specs=pl.BlockSpec((ADD_BLK, D), lambda i: (i, 0)),
        out_shape=jax.ShapeDtypeStruct((N_NODES, D), jnp.float32),
    )(a, b)


def kernel(edge_index, features, W1, b1, W2, b2, Wc):
    x_lin, zeros = _mlp(features, W1, b1, W2, b2, Wc)
    p0, p1 = _build_gcn_scatter()(edge_index, x_lin, zeros)
    return _add(p0, p1)


# speedup vs baseline: 10.2491x; 1.0072x over previous
import functools

import jax
import jax.numpy as jnp
import numpy as np
from jax import lax
from jax.experimental import pallas as pl
from jax.experimental.pallas import tpu as pltpu
from jax.experimental.pallas import tpu_sc as plsc

N_NODES = 10000
N_EDGES = 160000
DF = 256
DH = 512
D = 128

NC, NS = 2, 16
NW = NC * NS
CHUNK = 128
NCHUNK = 39
NXTRA = N_EDGES // CHUNK - NW * NCHUNK
ROWS_PT = 624
ROWS_LAST = N_NODES - 15 * ROWS_PT
ZROWS = 16

MLP_BLK = 5000
MLP_GRID = N_NODES // MLP_BLK
ADD_BLK = 5000
ADD_GRID = N_NODES // ADD_BLK


def _mlp_body(f_ref, w1_ref, b1_ref, w2_ref, b2_ref, wc_ref, o_ref, z_ref):
    z_ref[...] = jnp.zeros_like(z_ref)
    f = f_ref[...].astype(jnp.bfloat16)
    h = lax.dot_general(f, w1_ref[...].astype(jnp.bfloat16),
                        (((1,), (1,)), ((), ())),
                        preferred_element_type=jnp.float32)
    h = h + b1_ref[...][None, :]
    h = jnp.where(h >= 0, h, 0.01 * h)
    x = lax.dot_general(h.astype(jnp.bfloat16), w2_ref[...].astype(jnp.bfloat16),
                        (((1,), (1,)), ((), ())),
                        preferred_element_type=jnp.float32)
    x = x + b2_ref[...][None, :]
    o_ref[...] = jnp.dot(x, wc_ref[...], preferred_element_type=jnp.float32)


def _mlp(features, W1, b1, W2, b2, Wc):
    return pl.pallas_call(
        _mlp_body,
        grid=(MLP_GRID,),
        in_specs=[
            pl.BlockSpec((MLP_BLK, DF), lambda i: (i, 0)),
            pl.BlockSpec((DH, DF), lambda i: (0, 0)),
            pl.BlockSpec((DH,), lambda i: (0,)),
            pl.BlockSpec((D, DH), lambda i: (0, 0)),
            pl.BlockSpec((D,), lambda i: (0,)),
            pl.BlockSpec((D, D), lambda i: (0, 0)),
        ],
        out_specs=[pl.BlockSpec((MLP_BLK, D), lambda i: (i, 0)),
                   pl.BlockSpec((ROWS_LAST // MLP_GRID, D), lambda i: (i, 0))],
        out_shape=[jax.ShapeDtypeStruct((N_NODES, D), jnp.float32),
                   jax.ShapeDtypeStruct((ROWS_LAST, D), jnp.float32)],
    )(features, W1, b1, W2, b2, Wc)


@functools.cache
def _build_gcn_scatter():
  mesh = plsc.VectorSubcoreMesh(
      core_axis_name="c", subcore_axis_name="s", num_cores=NC, num_subcores=NS)

  nbuf = 2
  nck = NCHUNK + 1

  @functools.partial(
      pl.kernel,
      out_type=[jax.ShapeDtypeStruct((N_NODES, D), jnp.float32),
                jax.ShapeDtypeStruct((N_NODES, D), jnp.float32)],
      mesh=mesh,
      scratch_types=[
          pltpu.VMEM((NCHUNK * CHUNK,), jnp.int32),
          pltpu.VMEM((NCHUNK * CHUNK,), jnp.int32),
          pltpu.VMEM((CHUNK,), jnp.int32),
          pltpu.VMEM((CHUNK,), jnp.int32),
          [pltpu.VMEM((CHUNK, D), jnp.float32) for _ in range(nbuf)],
          pltpu.VMEM_SHARED((N_NODES, D), jnp.float32),
          [pltpu.SemaphoreType.DMA for _ in range(nbuf)],
      ],
  )
  def _gcn_scatter(ei_hbm, xlin_hbm, zeros_hbm, out0_hbm, out1_hbm,
                   src_v, dst_v, xsrc_v, xdst_v, bufs, acc, gsems):
    c = lax.axis_index("c")
    s = lax.axis_index("s")
    w = c * NS + s

    off = w * (NCHUNK * CHUNK)
    pltpu.sync_copy(ei_hbm.at[0, pl.ds(off, NCHUNK * CHUNK)], src_v)
    pltpu.sync_copy(ei_hbm.at[1, pl.ds(off, NCHUNK * CHUNK)], dst_v)

    @pl.when(s == 0)
    def _():
      xoff = (NW * NCHUNK + c) * CHUNK
      pltpu.sync_copy(ei_hbm.at[0, pl.ds(xoff, CHUNK)], xsrc_v)
      pltpu.sync_copy(ei_hbm.at[1, pl.ds(xoff, CHUNK)], xdst_v)

    def sidx(j):
      return src_v.at[pl.ds(j * CHUNK, CHUNK)]

    def didx(j):
      return dst_v.at[pl.ds(j * CHUNK, CHUNK)]

    def start_gather_ref(iref, b):
      pltpu.async_copy(xlin_hbm.at[iref], bufs[b], gsems[b])

    def wait_gather_ref(iref, b):
      pltpu.make_async_copy(xlin_hbm.at[iref], bufs[b], gsems[b]).wait()

    def scatter_ref(iref, b):
      pltpu.sync_copy(bufs[b], acc.at[iref], add=True)

    def start_gather(j, b):
      start_gather_ref(sidx(j), b)

    def wait_gather(j, b):
      wait_gather_ref(sidx(j), b)

    def scatter(j, b):
      scatter_ref(didx(j), b)

    for b in range(nbuf):
      start_gather(b, b)

    @pl.when(s < NS - 1)
    def _():
      pltpu.sync_copy(zeros_hbm.at[pl.ds(0, ROWS_PT)],
                      acc.at[pl.ds(s * ROWS_PT, ROWS_PT)])

    @pl.when(s == NS - 1)
    def _():
      pltpu.sync_copy(zeros_hbm.at[pl.ds(0, ROWS_LAST)],
                      acc.at[pl.ds(15 * ROWS_PT, ROWS_LAST)])

    plsc.subcore_barrier()

    ng = (NCHUNK - 1) // nbuf

    def body(g, carry):
      for b in range(nbuf):
        j = nbuf * g + b
        wait_gather(j, b)
        scatter(j, b)

        @pl.when(g < ng - 1)
        def _():
          start_gather(j + nbuf, b)
      return carry

    lax.fori_loop(0, ng, body, 0)

    start_gather(NCHUNK - 1, 0)

    @pl.when(s == 0)
    def _():
      start_gather_ref(xsrc_v, 1)

    wait_gather(NCHUNK - 1, 0)
    scatter(NCHUNK - 1, 0)

    @pl.when(s == 0)
    def _():
      wait_gather_ref(xsrc_v, 1)
      scatter_ref(xdst_v, 1)

    plsc.subcore_barrier()

    def writeout(out_hbm):
      @pl.when(s < NS - 1)
      def _():
        pltpu.sync_copy(acc.at[pl.ds(s * ROWS_PT, ROWS_PT)],
                        out_hbm.at[pl.ds(s * ROWS_PT, ROWS_PT)])

      @pl.when(s == NS - 1)
      def _():
        pltpu.sync_copy(acc.at[pl.ds(15 * ROWS_PT, ROWS_LAST)],
                        out_hbm.at[pl.ds(15 * ROWS_PT, ROWS_LAST)])

    @pl.when(c == 0)
    def _():
      writeout(out0_hbm)

    @pl.when(c == 1)
    def _():
      writeout(out1_hbm)

  return _gcn_scatter


def _add_body(a_ref, b_ref, o_ref):
    o_ref[...] = a_ref[...] + b_ref[...]


def _add(a, b):
    return pl.pallas_call(
        _add_body,
        grid=(ADD_GRID,),
        in_specs=[
            pl.BlockSpec((ADD_BLK, D), lambda i: (i, 0)),
            pl.BlockSpec((ADD_BLK, D), lambda i: (i, 0)),
        ],
        out_specs=pl.BlockSpec((ADD_BLK, D), lambda i: (i, 0)),
        out_shape=jax.ShapeDtypeStruct((N_NODES, D), jnp.float32),
    )(a, b)


def kernel(edge_index, features, W1, b1, W2, b2, Wc):
    x_lin, zeros = _mlp(features, W1, b1, W2, b2, Wc)
    p0, p1 = _build_gcn_scatter()(edge_index, x_lin, zeros)
    return _add(p0, p1)
